# baseline (speedup 1.0000x reference)
import functools

import jax
import jax.numpy as jnp
from jax import lax
from jax.experimental import pallas as pl
from jax.experimental.pallas import tpu as pltpu
from jax.experimental.pallas import tpu_sc as plsc

_NC = 2
_NS = 16
_NW = _NC * _NS
_LPW = 128


def _gather_body(table_hbm, idx_hbm, out_hbm, idx_v, idxt_v, gbuf, tbuf, *sems):
    gsems, osems = sems[:2], sems[2:]
    wid = lax.axis_index("s") * _NC + lax.axis_index("c")
    nf = idxt_v.shape[0]
    depth = gbuf.shape[2]
    per_w = nf * _LPW
    lanes = lax.iota(jnp.int32, 16)

    pltpu.sync_copy(idx_hbm.at[pl.ds(wid * per_w, per_w)], idx_v)

    @pl.loop(0, nf)
    def _tidx(f):
        @pl.loop(0, _LPW // 16)
        def _blk(lb):
            src = (lb * 16 + lanes) * nf + f
            idxt_v[f, pl.ds(lb * 16, 16)] = plsc.load_gather(idx_v, [src])

    pltpu.async_copy(table_hbm.at[idxt_v.at[0]], gbuf.at[0], gsems[0])
    pltpu.async_copy(table_hbm.at[idxt_v.at[1]], gbuf.at[1], gsems[1])

    @pl.loop(0, nf, step=2)
    def _fo(fo):
        for slot in range(2):
            f = fo + slot
            pltpu.make_async_copy(
                table_hbm.at[idxt_v.at[0]], gbuf.at[slot], gsems[slot]
            ).wait()

            @pl.when(f >= 2)
            def _():
                pltpu.make_async_copy(
                    tbuf.at[slot], out_hbm.at[0, :, 0], osems[slot]
                ).wait()

            for c in range(depth):
                a, s = c // 8, c % 8
                for lb in range(_LPW // 16):
                    rows = lb * 16 + lanes
                    cols = jnp.full((16,), c, jnp.int32)
                    tbuf[slot, a, s, pl.ds(lb * 16, 16)] = plsc.load_gather(
                        gbuf.at[slot], [rows, cols]
                    )

            pltpu.async_copy(tbuf.at[slot], out_hbm.at[f, :, wid], osems[slot])

            @pl.when(f + 2 < nf)
            def _():
                pltpu.async_copy(
                    table_hbm.at[idxt_v.at[f + 2]], gbuf.at[slot], gsems[slot]
                )

    pltpu.make_async_copy(tbuf.at[0], out_hbm.at[0, :, 0], osems[0]).wait()
    pltpu.make_async_copy(tbuf.at[1], out_hbm.at[0, :, 0], osems[1]).wait()


def kernel(x, weight):
    batch, fields = x.shape
    depth = weight.shape[1]
    total = batch * fields
    per_w = total // _NW
    ab = depth // 8
    tdim = batch // _LPW
    idx = x.reshape(total)

    call = pl.kernel(
        _gather_body,
        out_type=jax.ShapeDtypeStruct((fields, ab, tdim, 8, _LPW), jnp.float32),
        mesh=plsc.VectorSubcoreMesh(core_axis_name="c", subcore_axis_name="s"),
        scratch_types=[
            pltpu.VMEM((per_w,), jnp.int32),
            pltpu.VMEM((fields, _LPW), jnp.int32),
            pltpu.VMEM((2, _LPW, depth), jnp.float32),
            pltpu.VMEM((2, ab, 8, _LPW), jnp.float32),
        ] + [pltpu.SemaphoreType.DMA] * 4,
        compiler_params=pltpu.CompilerParams(
            use_tc_tiling_on_sc=False, needs_layout_passes=False
        ),
    )
    out5 = call(weight, idx)
    return out5.transpose(2, 4, 0, 1, 3).reshape(batch, fields, depth)


# speedup vs baseline: 1.5530x; 1.5530x over previous
import functools

import jax
import jax.numpy as jnp
from jax import lax
from jax.experimental import pallas as pl
from jax.experimental.pallas import tpu as pltpu
from jax.experimental.pallas import tpu_sc as plsc

_NC = 2
_NS = 16
_NW = _NC * _NS
_CH = 128
_NBUF = 2


def _gather_body(table_hbm, idx_hbm, out_hbm, idx_v, rows_v, *sems):
    gsems, osems = sems[:_NBUF], sems[_NBUF:]
    wid = lax.axis_index("s") * _NC + lax.axis_index("c")
    nchunk = idx_v.shape[0] // _CH
    base = wid * (nchunk * _CH)
    pltpu.sync_copy(idx_hbm.at[pl.ds(base, nchunk * _CH)], idx_v)

    for b in range(_NBUF):
        pltpu.async_copy(
            table_hbm.at[idx_v.at[pl.ds(b * _CH, _CH)]], rows_v.at[b], gsems[b]
        )

    @pl.loop(0, nchunk, step=_NBUF)
    def _outer(g):
        for b in range(_NBUF):
            j = g + b
            slot = b

            pltpu.make_async_copy(
                table_hbm.at[idx_v.at[pl.ds(0, _CH)]], rows_v.at[slot], gsems[slot]
            ).wait()
            pltpu.async_copy(
                rows_v.at[slot], out_hbm.at[pl.ds(base + j * _CH, _CH)], osems[slot]
            )

            @pl.when(j + _NBUF < nchunk)
            def _():
                pltpu.make_async_copy(
                    rows_v.at[slot], out_hbm.at[pl.ds(0, _CH)], osems[slot]
                ).wait()
                pltpu.async_copy(
                    table_hbm.at[idx_v.at[pl.ds((j + _NBUF) * _CH, _CH)]],
                    rows_v.at[slot],
                    gsems[slot],
                )

    for b in range(_NBUF):
        pltpu.make_async_copy(
            rows_v.at[b], out_hbm.at[pl.ds(0, _CH)], osems[b]
        ).wait()


def kernel(x, weight):
    batch, fields = x.shape
    depth = weight.shape[1]
    total = batch * fields
    per_w = total // _NW
    idx = x.reshape(total)

    call = pl.kernel(
        _gather_body,
        out_type=jax.ShapeDtypeStruct((total, depth), jnp.float32),
        mesh=plsc.VectorSubcoreMesh(core_axis_name="c", subcore_axis_name="s"),
        scratch_types=[
            pltpu.VMEM((per_w,), jnp.int32),
            pltpu.VMEM((_NBUF, _CH, depth), jnp.float32),
        ] + [pltpu.SemaphoreType.DMA] * (2 * _NBUF),
        compiler_params=pltpu.CompilerParams(use_tc_tiling_on_sc=False),
    )
    out = call(weight, idx)
    return out.reshape(batch, fields, depth)


# speedup vs baseline: 1.5953x; 1.0273x over previous
import functools

import jax
import jax.numpy as jnp
from jax import lax
from jax.experimental import pallas as pl
from jax.experimental.pallas import tpu as pltpu
from jax.experimental.pallas import tpu_sc as plsc

_NC = 2
_NS = 16
_NW = _NC * _NS
_CH = 128
_NBUF = 13


def _gather_body(table_hbm, idx_hbm, out_hbm, idx_v, rows_v, *sems):
    gsems, osems = sems[:_NBUF], sems[_NBUF:]
    wid = lax.axis_index("s") * _NC + lax.axis_index("c")
    nchunk = idx_v.shape[0] // _CH
    base = wid * (nchunk * _CH)
    pltpu.sync_copy(idx_hbm.at[pl.ds(base, nchunk * _CH)], idx_v)

    for b in range(_NBUF):
        pltpu.async_copy(
            table_hbm.at[idx_v.at[pl.ds(b * _CH, _CH)]], rows_v.at[b], gsems[b]
        )

    @pl.loop(0, nchunk, step=_NBUF)
    def _outer(g):
        for b in range(_NBUF):
            j = g + b
            slot = b

            pltpu.make_async_copy(
                table_hbm.at[idx_v.at[pl.ds(0, _CH)]], rows_v.at[slot], gsems[slot]
            ).wait()
            pltpu.async_copy(
                rows_v.at[slot], out_hbm.at[pl.ds(base + j * _CH, _CH)], osems[slot]
            )

            @pl.when(j + _NBUF < nchunk)
            def _():
                pltpu.make_async_copy(
                    rows_v.at[slot], out_hbm.at[pl.ds(0, _CH)], osems[slot]
                ).wait()
                pltpu.async_copy(
                    table_hbm.at[idx_v.at[pl.ds((j + _NBUF) * _CH, _CH)]],
                    rows_v.at[slot],
                    gsems[slot],
                )

    for b in range(_NBUF):
        pltpu.make_async_copy(
            rows_v.at[b], out_hbm.at[pl.ds(0, _CH)], osems[b]
        ).wait()


def kernel(x, weight):
    batch, fields = x.shape
    depth = weight.shape[1]
    total = batch * fields
    per_w = total // _NW
    idx = x.reshape(total)

    call = pl.kernel(
        _gather_body,
        out_type=jax.ShapeDtypeStruct((total, depth), jnp.float32),
        mesh=plsc.VectorSubcoreMesh(core_axis_name="c", subcore_axis_name="s"),
        scratch_types=[
            pltpu.VMEM((per_w,), jnp.int32),
            pltpu.VMEM((_NBUF, _CH, depth), jnp.float32),
        ] + [pltpu.SemaphoreType.DMA] * (2 * _NBUF),
        compiler_params=pltpu.CompilerParams(use_tc_tiling_on_sc=False),
    )
    out = call(weight, idx)
    return out.reshape(batch, fields, depth)
